# baseline (speedup 1.0000x reference)
import jax
import jax.numpy as jnp
from jax import lax
from jax.experimental import pallas as pl
from jax.experimental.pallas import tpu as pltpu
from jax.experimental.pallas import tpu_sc as plsc

B = 4096
S = 200
D = 64
SP = 208
NC = 2
NS = 16
NW = NC * NS
SEQ_W = B // NW
NG = SP // 16
NBUF = 3


def _rsqrt(x):
    i = lax.bitcast_convert_type(x, jnp.int32)
    i = jnp.int32(0x5F3759DF) - lax.shift_right_logical(i, 1)
    y = lax.bitcast_convert_type(i, jnp.float32)
    for _ in range(3):
        y = y * (1.5 - 0.5 * x * y * y)
    return y


def _allsum(v):
    for sh in (1, 2, 4, 8):
        perm = jnp.arange(16, dtype=jnp.int32) ^ sh
        v = v + jnp.take_along_axis(v, perm, axis=0)
    return v


def _lane_bcast(v, j):
    return jnp.take_along_axis(v, jnp.full((16,), j, jnp.int32), axis=0)


def _sc_body(ids_g_hbm, ids_m_hbm, table_hbm, pos_hbm, gb_hbm, out_hbm,
             ids_g_v, ids_m_v, pos_v, gb_v,
             emb0, emb1, emb2, gsem0, gsem1, gsem2, osem0, osem1, osem2):
    w = lax.axis_index("s") * NC + lax.axis_index("c")

    pltpu.sync_copy(ids_g_hbm.at[w], ids_g_v)
    pltpu.sync_copy(ids_m_hbm.at[w], ids_m_v)
    pltpu.sync_copy(pos_hbm, pos_v)
    pltpu.sync_copy(gb_hbm, gb_v)

    gvec = [gb_v[0, pl.ds(k * 16, 16)] for k in range(4)]
    bvec = [gb_v[1, pl.ds(k * 16, 16)] for k in range(4)]

    embs = (emb0, emb1, emb2)
    gsems = (gsem0, gsem1, gsem2)
    osems = (osem0, osem1, osem2)

    def start_gather(s, b):
        for h in range(2):
            pltpu.async_copy(table_hbm.at[ids_g_v.at[s, h]],
                             embs[b].at[pl.ds(h * 104, 104)], gsems[b])

    def wait_gather(s, b):
        for h in range(2):
            pltpu.make_async_copy(table_hbm.at[ids_g_v.at[s, h]],
                                  embs[b].at[pl.ds(h * 104, 104)],
                                  gsems[b]).wait()

    def start_out(s, b):
        pltpu.async_copy(embs[b].at[pl.ds(0, S)], out_hbm.at[w * SEQ_W + s],
                         osems[b])

    def wait_out(s, b):
        pltpu.make_async_copy(embs[b].at[pl.ds(0, S)],
                              out_hbm.at[w * SEQ_W + s], osems[b]).wait()

    def compute(s, b):
        emb = embs[b]

        def group_body(g, carry):
            ivec = ids_m_v[s, g]
            mvec = jnp.where(ivec != 0, jnp.float32(1.0), jnp.float32(0.0))
            for j in range(16):
                r = g * 16 + j
                m = _lane_bcast(mvec, j)
                x = [emb[r, pl.ds(k * 16, 16)] + pos_v[r, pl.ds(k * 16, 16)] * m
                     for k in range(4)]
                tot = _allsum(x[0] + x[1] + x[2] + x[3])
                sq = _allsum(x[0] * x[0] + x[1] * x[1]
                             + x[2] * x[2] + x[3] * x[3])
                mean = tot * (1.0 / 64.0)
                var = sq * (1.0 / 64.0) - mean * mean
                inv = _rsqrt(var + 1e-5)
                for k in range(4):
                    y = (x[k] - mean) * inv
                    emb[r, pl.ds(k * 16, 16)] = y * gvec[k] + bvec[k]
            return carry

        lax.fori_loop(0, NG, group_body, 0)

    start_gather(0, 0)

    def body(s, b, first):
        if not first:
            wait_out(s - 2, (b + 1) % NBUF)
        start_gather(s + 1, (b + 1) % NBUF)
        wait_gather(s, b)
        compute(s, b)
        start_out(s, b)

    body(0, 0, True)
    body(1, 1, True)
    body(2, 2, False)

    def loop_body(i, carry):
        s0 = 3 + 3 * i
        for off in range(3):
            body(s0 + off, off, False)
        return carry

    lax.fori_loop(0, (SEQ_W - 8) // 3, loop_body, 0)

    for s in range(SEQ_W - 5, SEQ_W - 1):
        body(s, s % NBUF, False)

    s_last = SEQ_W - 1
    wait_gather(s_last, s_last % NBUF)
    compute(s_last, s_last % NBUF)
    start_out(s_last, s_last % NBUF)
    for t in range(NBUF):
        s_d = SEQ_W - NBUF + t
        wait_out(s_d, s_d % NBUF)


def kernel(input_ids, table, pos_table, gamma, beta):
    ids = input_ids.astype(jnp.int32)
    ids_pad = jnp.pad(ids, ((0, 0), (0, SP - S)))
    ids_g = ids_pad.reshape(NW, SEQ_W, 2, 104)
    ids_m = ids_pad.reshape(NW, SEQ_W, NG, 16)
    pos_pad = jnp.pad(pos_table, ((0, SP - S), (0, 0)))
    gb = jnp.stack([gamma, beta])

    mesh = plsc.VectorSubcoreMesh(core_axis_name="c", subcore_axis_name="s")
    f = pl.kernel(
        _sc_body,
        out_type=jax.ShapeDtypeStruct((B, S, D), jnp.float32),
        mesh=mesh,
        compiler_params=pltpu.CompilerParams(use_tc_tiling_on_sc=False),
        scratch_types=[
            pltpu.VMEM((SEQ_W, 2, 104), jnp.int32),
            pltpu.VMEM((SEQ_W, NG, 16), jnp.int32),
            pltpu.VMEM((SP, D), jnp.float32),
            pltpu.VMEM((2, D), jnp.float32),
            pltpu.VMEM((SP, D), jnp.float32),
            pltpu.VMEM((SP, D), jnp.float32),
            pltpu.VMEM((SP, D), jnp.float32),
            pltpu.SemaphoreType.DMA,
            pltpu.SemaphoreType.DMA,
            pltpu.SemaphoreType.DMA,
            pltpu.SemaphoreType.DMA,
            pltpu.SemaphoreType.DMA,
            pltpu.SemaphoreType.DMA,
        ],
    )
    return f(ids_g, ids_m, table, pos_pad, gb)
